# baseline (speedup 1.0000x reference)
import functools

import jax
import jax.numpy as jnp
from jax import lax
from jax.experimental import pallas as pl
from jax.experimental.pallas import tpu as pltpu
from jax.experimental.pallas import tpu_sc as plsc

INPUT_SIZE = 64
BATCH = 16384
LANES = 16

_info = plsc.get_sparse_core_info()
NC = _info.num_cores
NS = _info.num_subcores
NW = NC * NS
BPW = BATCH // NW
CHUNK = 128
NCHUNK = BPW // CHUNK

_mesh = plsc.VectorSubcoreMesh(core_axis_name="c", subcore_axis_name="s")


@functools.partial(
    pl.kernel,
    mesh=_mesh,
    compiler_params=pltpu.CompilerParams(use_tc_tiling_on_sc=False),
    out_type=jax.ShapeDtypeStruct((BATCH, INPUT_SIZE), jnp.float32),
    scratch_types=[
        pltpu.VMEM((NCHUNK, CHUNK), jnp.int32),
        pltpu.VMEM((NCHUNK, CHUNK), jnp.int32),
        pltpu.VMEM((BPW, INPUT_SIZE), jnp.float32),
        pltpu.VMEM((BPW, INPUT_SIZE), jnp.float32),
        pltpu.SemaphoreType.DMA,
        pltpu.SemaphoreType.DMA,
        pltpu.SemaphoreType.DMA,
    ],
)
def _aux_encode(x_hbm, ind_hbm, w_hbm, map_hbm, out_hbm,
                idx_v, mapped_v, rows_v, x_v, sem_m, sem_w, sem_x):
    wid = lax.axis_index("s") * NC + lax.axis_index("c")
    base = wid * BPW

    pltpu.sync_copy(ind_hbm.at[wid], idx_v)

    cp_x = pltpu.async_copy(x_hbm.at[pl.ds(base, BPW)], x_v, sem_x)

    map_cps = [
        pltpu.async_copy(map_hbm.at[idx_v.at[c]], mapped_v.at[c], sem_m)
        for c in range(NCHUNK)
    ]
    row_cps = []
    for c in range(NCHUNK):
        map_cps[c].wait()
        row_cps.append(
            pltpu.async_copy(
                w_hbm.at[mapped_v.at[c]],
                rows_v.at[pl.ds(c * CHUNK, CHUNK)],
                sem_w,
            )
        )
    for cp in row_cps:
        cp.wait()
    cp_x.wait()

    def body(i, _):
        for j in range(INPUT_SIZE // LANES):
            sl = pl.ds(j * LANES, LANES)
            x_v[i, sl] = x_v[i, sl] + rows_v[i, sl]
        return 0

    lax.fori_loop(0, BPW, body, 0)

    pltpu.sync_copy(x_v, out_hbm.at[pl.ds(base, BPW)])


def kernel(x, ind, weight, mapping):
    ind32 = ind.astype(jnp.int32).reshape(NW, NCHUNK, CHUNK)
    map32 = mapping.astype(jnp.int32)
    return _aux_encode(x, ind32, weight, map32)


# speedup vs baseline: 1.5925x; 1.5925x over previous
import functools

import jax
import jax.numpy as jnp
from jax import lax
from jax.experimental import pallas as pl
from jax.experimental.pallas import tpu as pltpu
from jax.experimental.pallas import tpu_sc as plsc

INPUT_SIZE = 64
BATCH = 16384
LANES = 16

_info = plsc.get_sparse_core_info()
NC = _info.num_cores
NS = _info.num_subcores
NW = NC * NS
BPW = BATCH // NW
CHUNK = 128
NCHUNK = BPW // CHUNK
FPW = BPW * INPUT_SIZE

_mesh = plsc.VectorSubcoreMesh(core_axis_name="c", subcore_axis_name="s")


@functools.partial(
    pl.kernel,
    mesh=_mesh,
    out_type=jax.ShapeDtypeStruct((BATCH * INPUT_SIZE,), jnp.float32),
    scratch_types=[
        pltpu.VMEM((BPW,), jnp.int32),
        pltpu.VMEM((BPW,), jnp.int32),
        pltpu.VMEM((BPW, INPUT_SIZE), jnp.float32),
        pltpu.VMEM((FPW,), jnp.float32),
        pltpu.SemaphoreType.DMA,
        pltpu.SemaphoreType.DMA,
        pltpu.SemaphoreType.DMA,
    ],
)
def _aux_encode(x_hbm, ind_hbm, w_hbm, map_hbm, out_hbm,
                idx_v, mapped_v, rows_v, x_v, sem_m, sem_w, sem_x):
    wid = lax.axis_index("s") * NC + lax.axis_index("c")
    base = wid * BPW

    pltpu.sync_copy(ind_hbm.at[pl.ds(base, BPW)], idx_v)

    cp_x = pltpu.async_copy(x_hbm.at[pl.ds(wid * FPW, FPW)], x_v, sem_x)

    map_cps = [
        pltpu.async_copy(
            map_hbm.at[idx_v.at[pl.ds(c * CHUNK, CHUNK)]],
            mapped_v.at[pl.ds(c * CHUNK, CHUNK)],
            sem_m,
        )
        for c in range(NCHUNK)
    ]
    for cp in map_cps:
        cp.wait()

    def fire(g, _):
        vec = mapped_v[pl.ds(g * LANES, LANES)]
        for j in range(LANES):
            pltpu.async_copy(w_hbm.at[vec[j]], rows_v.at[g * LANES + j], sem_w)
        return 0

    lax.fori_loop(0, BPW // LANES, fire, 0)

    def drain(i, _):
        pltpu.make_async_copy(w_hbm.at[0], rows_v.at[i], sem_w).wait()
        return 0

    lax.fori_loop(0, BPW, drain, 0)
    cp_x.wait()

    def body(k, _):
        row = k // (INPUT_SIZE // LANES)
        col = (k % (INPUT_SIZE // LANES)) * LANES
        sl = pl.ds(k * LANES, LANES)
        x_v[sl] = x_v[sl] + rows_v[row, pl.ds(col, LANES)]
        return 0

    lax.fori_loop(0, FPW // LANES, body, 0)

    pltpu.sync_copy(x_v, out_hbm.at[pl.ds(wid * FPW, FPW)])


def kernel(x, ind, weight, mapping):
    out_flat = _aux_encode(
        x.reshape(-1),
        ind.astype(jnp.int32),
        weight,
        mapping.astype(jnp.int32),
    )
    return out_flat.reshape(BATCH, INPUT_SIZE)
